# baseline (speedup 1.0000x reference)
import jax
import jax.numpy as jnp
from jax import lax
from jax.experimental import pallas as pl
from jax.experimental.pallas import tpu as pltpu
from jax.experimental.pallas import tpu_sc as plsc

_CHUNK = 128
_LANES = 128
_TBLOCK = 2048


def _padT_block(x_ref, o_ref):
    feat = x_ref.shape[0]
    o_ref[:, :feat] = x_ref[...].T


def _pad_transpose(emb_t):
    feat, num = emb_t.shape
    grid = (num + _TBLOCK - 1) // _TBLOCK
    return pl.pallas_call(
        _padT_block,
        grid=(grid,),
        in_specs=[pl.BlockSpec((feat, _TBLOCK), lambda i: (0, i))],
        out_specs=pl.BlockSpec((_TBLOCK, _LANES), lambda i: (i, 0)),
        out_shape=jax.ShapeDtypeStruct((num, _LANES), emb_t.dtype),
        compiler_params=pltpu.CompilerParams(
            dimension_semantics=("parallel",)),
    )(emb_t)


def kernel(input_ids, embedding):
    batch, hist = input_ids.shape
    num_idx = batch * hist
    num_emb, features = embedding.shape

    info = plsc.get_sparse_core_info()
    n_workers = info.num_cores * info.num_subcores
    per_w = num_idx // n_workers
    n_chunks = per_w // _CHUNK

    idx = input_ids.reshape(n_workers, n_chunks, _CHUNK).astype(jnp.int32)
    emb_wide = _pad_transpose(embedding.T)

    mesh = plsc.VectorSubcoreMesh(core_axis_name="c", subcore_axis_name="s")

    @jax.jit
    def gather(emb, ids):
        @pl.kernel(
            out_type=jax.ShapeDtypeStruct((num_idx, _LANES), emb.dtype),
            mesh=mesh,
            scratch_types=[
                pltpu.VMEM((n_chunks, _CHUNK), jnp.int32),
                pltpu.VMEM((4, _CHUNK, _LANES), jnp.float32),
                pltpu.SemaphoreType.DMA((4,)),
                pltpu.SemaphoreType.DMA((4,)),
            ],
        )
        def gather_kernel(emb_hbm, idx_hbm, out_hbm, idx_v, bufs, gsem, wsem):
            wid = lax.axis_index("s") * info.num_cores + lax.axis_index("c")
            base = wid * per_w
            pltpu.sync_copy(idx_hbm.at[wid], idx_v)

            def g_start(c, b):
                pltpu.async_copy(
                    emb_hbm.at[idx_v.at[c]], bufs.at[b], gsem.at[b])

            def g_drain(b):
                pltpu.make_async_copy(
                    emb_hbm.at[idx_v.at[0]], bufs.at[b], gsem.at[b]).wait()

            def w_start(c, b):
                pltpu.async_copy(
                    bufs.at[b],
                    out_hbm.at[pl.ds(base + c * _CHUNK, _CHUNK)],
                    wsem.at[b])

            def w_drain(b):
                pltpu.make_async_copy(
                    bufs.at[b], out_hbm.at[pl.ds(base, _CHUNK)],
                    wsem.at[b]).wait()

            g_start(0, 0)
            g_start(1, 1)

            @pl.loop(0, n_chunks - 4, step=4)
            def _(g):
                g_drain(0)
                w_start(g + 0, 0)
                g_drain(1)
                w_start(g + 1, 1)

                @pl.when(g > 0)
                def _():
                    w_drain(2)
                    w_drain(3)

                g_start(g + 2, 2)
                g_start(g + 3, 3)
                g_drain(2)
                w_start(g + 2, 2)
                g_drain(3)
                w_start(g + 3, 3)
                w_drain(0)
                w_drain(1)
                g_start(g + 4, 0)
                g_start(g + 5, 1)

            e = n_chunks - 4
            g_drain(0)
            w_start(e + 0, 0)
            g_drain(1)
            w_start(e + 1, 1)
            w_drain(2)
            w_drain(3)
            g_start(e + 2, 2)
            g_start(e + 3, 3)
            g_drain(2)
            w_start(e + 2, 2)
            g_drain(3)
            w_start(e + 3, 3)
            w_drain(0)
            w_drain(1)
            w_drain(2)
            w_drain(3)

        return gather_kernel(emb, ids)

    out = gather(emb_wide, idx)
    return out[:, :features].reshape(batch, hist, features)


# speedup vs baseline: 1.2658x; 1.2658x over previous
import jax
import jax.numpy as jnp
from jax import lax
from jax.experimental import pallas as pl
from jax.experimental.pallas import tpu as pltpu
from jax.experimental.pallas import tpu_sc as plsc

_CHUNK = 128
_LANES = 128
_TBLOCK = 8192


def _padT_block(x_ref, o_ref):
    feat = x_ref.shape[0]
    o_ref[:, :feat] = x_ref[...].T


def _pad_transpose(emb_t):
    feat, num = emb_t.shape
    grid = (num + _TBLOCK - 1) // _TBLOCK
    return pl.pallas_call(
        _padT_block,
        grid=(grid,),
        in_specs=[pl.BlockSpec((feat, _TBLOCK), lambda i: (0, i))],
        out_specs=pl.BlockSpec((_TBLOCK, _LANES), lambda i: (i, 0)),
        out_shape=jax.ShapeDtypeStruct((num, _LANES), emb_t.dtype),
        compiler_params=pltpu.CompilerParams(
            dimension_semantics=("parallel",)),
    )(emb_t)


def kernel(input_ids, embedding):
    batch, hist = input_ids.shape
    num_idx = batch * hist
    num_emb, features = embedding.shape

    info = plsc.get_sparse_core_info()
    n_workers = info.num_cores * info.num_subcores
    per_w = num_idx // n_workers
    n_chunks = per_w // _CHUNK

    idx = input_ids.reshape(n_workers, n_chunks, _CHUNK).astype(jnp.int32)
    emb_wide = _pad_transpose(embedding.T)

    mesh = plsc.VectorSubcoreMesh(core_axis_name="c", subcore_axis_name="s")

    @jax.jit
    def gather(emb, ids):
        @pl.kernel(
            out_type=jax.ShapeDtypeStruct((num_idx, _LANES), emb.dtype),
            mesh=mesh,
            scratch_types=[
                pltpu.VMEM((n_chunks, _CHUNK), jnp.int32),
                pltpu.VMEM((4, _CHUNK, _LANES), jnp.float32),
                pltpu.SemaphoreType.DMA((4,)),
                pltpu.SemaphoreType.DMA((4,)),
            ],
        )
        def gather_kernel(emb_hbm, idx_hbm, out_hbm, idx_v, bufs, gsem, wsem):
            wid = lax.axis_index("s") * info.num_cores + lax.axis_index("c")
            base = wid * per_w
            pltpu.sync_copy(idx_hbm.at[wid], idx_v)

            def g_start(c, b):
                pltpu.async_copy(
                    emb_hbm.at[idx_v.at[c]], bufs.at[b], gsem.at[b])

            def g_drain(b):
                pltpu.make_async_copy(
                    emb_hbm.at[idx_v.at[0]], bufs.at[b], gsem.at[b]).wait()

            def w_start(c, b):
                pltpu.async_copy(
                    bufs.at[b],
                    out_hbm.at[pl.ds(base + c * _CHUNK, _CHUNK)],
                    wsem.at[b])

            def w_drain(b):
                pltpu.make_async_copy(
                    bufs.at[b], out_hbm.at[pl.ds(base, _CHUNK)],
                    wsem.at[b]).wait()

            g_start(0, 0)
            g_start(1, 1)

            @pl.loop(0, n_chunks - 4, step=4)
            def _(g):
                g_drain(0)
                w_start(g + 0, 0)
                g_drain(1)
                w_start(g + 1, 1)

                @pl.when(g > 0)
                def _():
                    w_drain(2)
                    w_drain(3)

                g_start(g + 2, 2)
                g_start(g + 3, 3)
                g_drain(2)
                w_start(g + 2, 2)
                g_drain(3)
                w_start(g + 3, 3)
                w_drain(0)
                w_drain(1)
                g_start(g + 4, 0)
                g_start(g + 5, 1)

            e = n_chunks - 4
            g_drain(0)
            w_start(e + 0, 0)
            g_drain(1)
            w_start(e + 1, 1)
            w_drain(2)
            w_drain(3)
            g_start(e + 2, 2)
            g_start(e + 3, 3)
            g_drain(2)
            w_start(e + 2, 2)
            g_drain(3)
            w_start(e + 3, 3)
            w_drain(0)
            w_drain(1)
            w_drain(2)
            w_drain(3)

        return gather_kernel(emb, ids)

    out = gather(emb_wide, idx)
    return out[:, :features].reshape(batch, hist, features)


# speedup vs baseline: 1.2980x; 1.0255x over previous
import jax
import jax.numpy as jnp
from jax import lax
from jax.experimental import pallas as pl
from jax.experimental.pallas import tpu as pltpu
from jax.experimental.pallas import tpu_sc as plsc

_CHUNK = 128
_LANES = 128
_TBLOCK = 16384


def _padT_block(x_ref, o_ref):
    feat = x_ref.shape[0]
    o_ref[:, :feat] = x_ref[...].T


def _pad_transpose(emb_t):
    feat, num = emb_t.shape
    grid = (num + _TBLOCK - 1) // _TBLOCK
    return pl.pallas_call(
        _padT_block,
        grid=(grid,),
        in_specs=[pl.BlockSpec((feat, _TBLOCK), lambda i: (0, i))],
        out_specs=pl.BlockSpec((_TBLOCK, _LANES), lambda i: (i, 0)),
        out_shape=jax.ShapeDtypeStruct((num, _LANES), emb_t.dtype),
        compiler_params=pltpu.CompilerParams(
            dimension_semantics=("parallel",)),
    )(emb_t)


def kernel(input_ids, embedding):
    batch, hist = input_ids.shape
    num_idx = batch * hist
    num_emb, features = embedding.shape

    info = plsc.get_sparse_core_info()
    n_workers = info.num_cores * info.num_subcores
    per_w = num_idx // n_workers
    n_chunks = per_w // _CHUNK

    idx = input_ids.reshape(n_workers, n_chunks, _CHUNK).astype(jnp.int32)
    emb_wide = _pad_transpose(embedding.T)

    mesh = plsc.VectorSubcoreMesh(core_axis_name="c", subcore_axis_name="s")

    @jax.jit
    def gather(emb, ids):
        @pl.kernel(
            out_type=jax.ShapeDtypeStruct((num_idx, _LANES), emb.dtype),
            mesh=mesh,
            scratch_types=[
                pltpu.VMEM((n_chunks, _CHUNK), jnp.int32),
                pltpu.VMEM((4, _CHUNK, _LANES), jnp.float32),
                pltpu.SemaphoreType.DMA((4,)),
                pltpu.SemaphoreType.DMA((4,)),
            ],
        )
        def gather_kernel(emb_hbm, idx_hbm, out_hbm, idx_v, bufs, gsem, wsem):
            wid = lax.axis_index("s") * info.num_cores + lax.axis_index("c")
            base = wid * per_w
            pltpu.sync_copy(idx_hbm.at[wid], idx_v)

            def g_start(c, b):
                pltpu.async_copy(
                    emb_hbm.at[idx_v.at[c]], bufs.at[b], gsem.at[b])

            def g_drain(b):
                pltpu.make_async_copy(
                    emb_hbm.at[idx_v.at[0]], bufs.at[b], gsem.at[b]).wait()

            def w_start(c, b):
                pltpu.async_copy(
                    bufs.at[b],
                    out_hbm.at[pl.ds(base + c * _CHUNK, _CHUNK)],
                    wsem.at[b])

            def w_drain(b):
                pltpu.make_async_copy(
                    bufs.at[b], out_hbm.at[pl.ds(base, _CHUNK)],
                    wsem.at[b]).wait()

            g_start(0, 0)
            g_start(1, 1)

            @pl.loop(0, n_chunks - 4, step=4)
            def _(g):
                g_drain(0)
                w_start(g + 0, 0)
                g_drain(1)
                w_start(g + 1, 1)

                @pl.when(g > 0)
                def _():
                    w_drain(2)
                    w_drain(3)

                g_start(g + 2, 2)
                g_start(g + 3, 3)
                g_drain(2)
                w_start(g + 2, 2)
                g_drain(3)
                w_start(g + 3, 3)
                w_drain(0)
                w_drain(1)
                g_start(g + 4, 0)
                g_start(g + 5, 1)

            e = n_chunks - 4
            g_drain(0)
            w_start(e + 0, 0)
            g_drain(1)
            w_start(e + 1, 1)
            w_drain(2)
            w_drain(3)
            g_start(e + 2, 2)
            g_start(e + 3, 3)
            g_drain(2)
            w_start(e + 2, 2)
            g_drain(3)
            w_start(e + 3, 3)
            w_drain(0)
            w_drain(1)
            w_drain(2)
            w_drain(3)

        return gather_kernel(emb, ids)

    out = gather(emb_wide, idx)
    return out[:, :features].reshape(batch, hist, features)


# speedup vs baseline: 1.3104x; 1.0095x over previous
import jax
import jax.numpy as jnp
from jax import lax
from jax.experimental import pallas as pl
from jax.experimental.pallas import tpu as pltpu
from jax.experimental.pallas import tpu_sc as plsc

_CHUNK = 128
_LANES = 128
_TBLOCK = 32768


def _padT_block(x_ref, o_ref):
    feat = x_ref.shape[0]
    o_ref[:, :feat] = x_ref[...].T


def _pad_transpose(emb_t):
    feat, num = emb_t.shape
    grid = (num + _TBLOCK - 1) // _TBLOCK
    return pl.pallas_call(
        _padT_block,
        grid=(grid,),
        in_specs=[pl.BlockSpec((feat, _TBLOCK), lambda i: (0, i))],
        out_specs=pl.BlockSpec((_TBLOCK, _LANES), lambda i: (i, 0)),
        out_shape=jax.ShapeDtypeStruct((num, _LANES), emb_t.dtype),
        compiler_params=pltpu.CompilerParams(
            dimension_semantics=("parallel",)),
    )(emb_t)


def kernel(input_ids, embedding):
    batch, hist = input_ids.shape
    num_idx = batch * hist
    num_emb, features = embedding.shape

    info = plsc.get_sparse_core_info()
    n_workers = info.num_cores * info.num_subcores
    per_w = num_idx // n_workers
    n_chunks = per_w // _CHUNK

    idx = input_ids.reshape(n_workers, n_chunks, _CHUNK).astype(jnp.int32)
    emb_wide = _pad_transpose(embedding.T)

    mesh = plsc.VectorSubcoreMesh(core_axis_name="c", subcore_axis_name="s")

    @jax.jit
    def gather(emb, ids):
        @pl.kernel(
            out_type=jax.ShapeDtypeStruct((num_idx, _LANES), emb.dtype),
            mesh=mesh,
            scratch_types=[
                pltpu.VMEM((n_chunks, _CHUNK), jnp.int32),
                pltpu.VMEM((4, _CHUNK, _LANES), jnp.float32),
                pltpu.SemaphoreType.DMA((4,)),
                pltpu.SemaphoreType.DMA((4,)),
            ],
        )
        def gather_kernel(emb_hbm, idx_hbm, out_hbm, idx_v, bufs, gsem, wsem):
            wid = lax.axis_index("s") * info.num_cores + lax.axis_index("c")
            base = wid * per_w
            pltpu.sync_copy(idx_hbm.at[wid], idx_v)

            def g_start(c, b):
                pltpu.async_copy(
                    emb_hbm.at[idx_v.at[c]], bufs.at[b], gsem.at[b])

            def g_drain(b):
                pltpu.make_async_copy(
                    emb_hbm.at[idx_v.at[0]], bufs.at[b], gsem.at[b]).wait()

            def w_start(c, b):
                pltpu.async_copy(
                    bufs.at[b],
                    out_hbm.at[pl.ds(base + c * _CHUNK, _CHUNK)],
                    wsem.at[b])

            def w_drain(b):
                pltpu.make_async_copy(
                    bufs.at[b], out_hbm.at[pl.ds(base, _CHUNK)],
                    wsem.at[b]).wait()

            g_start(0, 0)
            g_start(1, 1)

            @pl.loop(0, n_chunks - 4, step=4)
            def _(g):
                g_drain(0)
                w_start(g + 0, 0)
                g_drain(1)
                w_start(g + 1, 1)

                @pl.when(g > 0)
                def _():
                    w_drain(2)
                    w_drain(3)

                g_start(g + 2, 2)
                g_start(g + 3, 3)
                g_drain(2)
                w_start(g + 2, 2)
                g_drain(3)
                w_start(g + 3, 3)
                w_drain(0)
                w_drain(1)
                g_start(g + 4, 0)
                g_start(g + 5, 1)

            e = n_chunks - 4
            g_drain(0)
            w_start(e + 0, 0)
            g_drain(1)
            w_start(e + 1, 1)
            w_drain(2)
            w_drain(3)
            g_start(e + 2, 2)
            g_start(e + 3, 3)
            g_drain(2)
            w_start(e + 2, 2)
            g_drain(3)
            w_start(e + 3, 3)
            w_drain(0)
            w_drain(1)
            w_drain(2)
            w_drain(3)

        return gather_kernel(emb, ids)

    out = gather(emb_wide, idx)
    return out[:, :features].reshape(batch, hist, features)
